# baseline (speedup 1.0000x reference)
import functools

import jax
import jax.numpy as jnp
from jax import lax
from jax.experimental import pallas as pl
from jax.experimental.pallas import tpu as pltpu
from jax.experimental.pallas import tpu_sc as plsc

VOCAB = 128100
BATCH = 128
SEQ = 512
DIM = 1024
EPS = 1e-07

NC = 2
NS = 16
NW = NC * NS
SBLK = SEQ // NW
NB = 4
NCHUNK = BATCH // NB
NJ = DIM // 16


def _rsqrt16(x):
    i = lax.bitcast_convert_type(x, jnp.int32)
    i = jnp.int32(0x5F3759DF) - (i >> 1)
    y = lax.bitcast_convert_type(i, jnp.float32)
    half = x * 0.5
    for _ in range(3):
        y = y * (1.5 - half * y * y)
    return y


def _body(ids_hbm, w_hbm, pos_hbm, g_hbm, b_hbm, out_hbm,
          idx_v, pos_v, g_v, b_v, rows_v, sem):
    wid = lax.axis_index("s") * NC + lax.axis_index("c")
    s0 = wid * SBLK

    pltpu.sync_copy(ids_hbm.at[wid], idx_v)
    pltpu.sync_copy(pos_hbm.at[pl.ds(s0, SBLK), :], pos_v)
    pltpu.sync_copy(g_hbm, g_v)
    pltpu.sync_copy(b_hbm, b_v)

    inv_d = jnp.float32(1.0 / DIM)

    @pl.loop(0, NCHUNK)
    def _chunk(c):
        b0 = c * NB
        for b in range(NB):
            pltpu.async_copy(w_hbm.at[idx_v.at[pl.ds((b0 + b) * SBLK, SBLK)]],
                             rows_v.at[b], sem)
        for b in range(NB):
            pltpu.make_async_copy(
                w_hbm.at[idx_v.at[pl.ds((b0 + b) * SBLK, SBLK)]],
                rows_v.at[b], sem).wait()

        @pl.loop(0, NB * SBLK)
        def _row(i):
            b = i // SBLK
            si = i % SBLK

            def j1(j, carry):
                s, s2 = carry
                x = rows_v[b, si, pl.ds(j * 16, 16)] + pos_v[si, pl.ds(j * 16, 16)]
                rows_v[b, si, pl.ds(j * 16, 16)] = x
                return (s + x, s2 + x * x)

            z = jnp.zeros((16,), jnp.float32)
            s, s2 = lax.fori_loop(0, NJ, j1, (z, z), unroll=4)
            lanes = lax.iota(jnp.int32, 16)
            for sh in (8, 4, 2, 1):
                perm = lanes ^ sh
                s = s + s.at[perm].get(mode="promise_in_bounds")
                s2 = s2 + s2.at[perm].get(mode="promise_in_bounds")
            mean_v = s * inv_d
            var_v = s2 * inv_d - mean_v * mean_v
            rstd_v = _rsqrt16(var_v + EPS)

            def j2(j, _):
                x = rows_v[b, si, pl.ds(j * 16, 16)]
                g = g_v[pl.ds(j * 16, 16)]
                bt = b_v[pl.ds(j * 16, 16)]
                rows_v[b, si, pl.ds(j * 16, 16)] = (x - mean_v) * rstd_v * g + bt
                return 0

            lax.fori_loop(0, NJ, j2, 0, unroll=4)

        pltpu.sync_copy(rows_v, out_hbm.at[pl.ds(b0, NB), pl.ds(s0, SBLK), :])


_sc_call = functools.partial(
    pl.kernel,
    out_type=jax.ShapeDtypeStruct((BATCH, SEQ, DIM), jnp.float32),
    mesh=plsc.VectorSubcoreMesh(core_axis_name="c", subcore_axis_name="s",
                                num_cores=NC, num_subcores=NS),
    scratch_types=[
        pltpu.VMEM((BATCH * SBLK,), jnp.int32),
        pltpu.VMEM((SBLK, DIM), jnp.float32),
        pltpu.VMEM((DIM,), jnp.float32),
        pltpu.VMEM((DIM,), jnp.float32),
        pltpu.VMEM((NB, SBLK, DIM), jnp.float32),
        pltpu.SemaphoreType.DMA,
    ],
)(_body)


def kernel(input_ids, weight, position_embeddings, ln_gamma, ln_beta):
    ids_perm = (input_ids.astype(jnp.int32)
                .reshape(BATCH, NW, SBLK)
                .transpose(1, 0, 2)
                .reshape(NW, BATCH * SBLK))
    return _sc_call(ids_perm, weight, position_embeddings, ln_gamma, ln_beta)
